# Initial kernel scaffold; baseline (speedup 1.0000x reference)
import jax
import jax.numpy as jnp
from jax.experimental import pallas as pl


def kernel(x, edge_index, edge_weight, convW, convB, lnG, lnB, resW, resB, resLnG, resLnB):
    raise NotImplementedError("write your pallas kernel here")


# speedup vs baseline: 5.7603x; 5.7603x over previous
import functools

import jax
import jax.numpy as jnp
import numpy as np
from jax import lax
from jax.experimental import pallas as pl
from jax.experimental.pallas import tpu as pltpu
from jax.experimental.pallas import tpu_sc as plsc

N = 10000
E = 320000
D = 128
H = D // 2
NC = 4
NP = 10240
K = 112
CH = 180
KD = 80
CHD = 126
EP = 16 * CH * K
SEG = NP // 16
ZR = 128
R = 1280
EPS = 1e-5

_mesh = plsc.VectorSubcoreMesh(core_axis_name="c", subcore_axis_name="s")
_sc_params = pltpu.CompilerParams(use_tc_tiling_on_sc=False)


@functools.partial(
    pl.kernel,
    out_type=jax.ShapeDtypeStruct((2, NP), jnp.float32),
    mesh=_mesh,
    scratch_types=[
        pltpu.VMEM((CHD, KD), jnp.int32),
        pltpu.VMEM((CHD, KD), jnp.float32),
        pltpu.VMEM((SEG,), jnp.float32),
        pltpu.VMEM_SHARED((NP,), jnp.float32),
    ],
    compiler_params=_sc_params,
)
def _sc_deg(cols, ws, out, col_v, w_v, zv, dacc):
    cid = lax.axis_index("c")
    sid = lax.axis_index("s")
    wid = sid * 2 + cid
    pltpu.sync_copy(cols.at[wid], col_v)
    pltpu.sync_copy(ws.at[wid], w_v)

    z16 = jnp.zeros((16,), jnp.float32)

    @pl.loop(0, SEG // 16)
    def _(r):
        zv[pl.ds(r * 16, 16)] = z16

    pltpu.sync_copy(zv, dacc.at[pl.ds(sid * SEG, SEG)])
    plsc.subcore_barrier()

    @pl.loop(0, CHD)
    def _(g):
        pltpu.sync_copy(w_v.at[g], dacc.at[col_v.at[g]], add=True)

    plsc.subcore_barrier()
    pltpu.sync_copy(dacc.at[pl.ds(sid * SEG, SEG)],
                    out.at[cid, pl.ds(sid * SEG, SEG)])


@functools.partial(
    pl.kernel,
    out_type=jax.ShapeDtypeStruct((2, NP, H), jnp.float32),
    mesh=_mesh,
    scratch_types=[
        pltpu.VMEM((CH, K), jnp.int32),
        pltpu.VMEM((CH, K), jnp.int32),
        pltpu.VMEM((CH, K), jnp.float32),
        pltpu.VMEM((2, K, H), jnp.float32),
        pltpu.VMEM((ZR, H), jnp.float32),
        pltpu.VMEM_SHARED((NP, H), jnp.float32),
        pltpu.SemaphoreType.DMA,
        pltpu.SemaphoreType.DMA,
    ],
    compiler_params=_sc_params,
)
def _sc_layer(hw2v, rows2, cols, ws, out, row_v, col_v, w_v, msg, zbuf, acc,
              sem0, sem1):
    cid = lax.axis_index("c")
    sid = lax.axis_index("s")
    pltpu.sync_copy(rows2.at[cid, sid], row_v)
    pltpu.sync_copy(cols.at[sid], col_v)
    pltpu.sync_copy(ws.at[sid], w_v)

    z16 = jnp.zeros((16,), jnp.float32)

    @pl.loop(0, ZR)
    def _(r):
        for d in range(H // 16):
            zbuf[r, pl.ds(d * 16, 16)] = z16

    for j in range(SEG // ZR):
        pltpu.sync_copy(zbuf, acc.at[pl.ds(sid * SEG + j * ZR, ZR)])
    plsc.subcore_barrier()

    sems = (sem0, sem1)
    for b in range(2):
        pltpu.async_copy(hw2v.at[row_v.at[b]], msg.at[b], sems[b])

    @pl.loop(0, CH, step=2)
    def _(go):
        for b in range(2):
            g = go + b
            pltpu.make_async_copy(hw2v.at[row_v.at[g]], msg.at[b],
                                  sems[b]).wait()

            @pl.loop(0, K // 16)
            def _(jg):
                w16 = w_v[g, pl.ds(jg * 16, 16)]
                eb = jg * 16
                for el in range(16):
                    wspl = jnp.full((16,), w16[el])
                    for d in range(H // 16):
                        sl = pl.ds(d * 16, 16)
                        msg[b, eb + el, sl] = msg[b, eb + el, sl] * wspl

            pltpu.sync_copy(msg.at[b], acc.at[col_v.at[g]], add=True)

            @pl.when(g + 2 < CH)
            def _():
                pltpu.async_copy(hw2v.at[row_v.at[g + 2]], msg.at[b], sems[b])

    plsc.subcore_barrier()
    pltpu.sync_copy(acc.at[pl.ds(sid * SEG, SEG)],
                    out.at[cid, pl.ds(sid * SEG, SEG)])


def _ln(h, g, b):
    mu = jnp.mean(h, axis=-1, keepdims=True)
    d = h - mu
    var = jnp.mean(d * d, axis=-1, keepdims=True)
    return d / jnp.sqrt(var + EPS) * g + b


def _gelu(x):
    return 0.5 * x * (1.0 + lax.erf(x * np.float32(1.0 / np.sqrt(2.0))))


def _tc_pre_body(x_ref, degp_ref, resW_ref, resB_ref, resG_ref, resBe_ref,
                 W0_ref, id_ref, hw2_ref, dis_ref):
    x = x_ref[...]
    deg = degp_ref[0] + degp_ref[1]
    dis = jnp.where(deg > 0, lax.rsqrt(jnp.where(deg > 0, deg, 1.0)), 0.0)
    dis_ref[...] = dis
    t = jnp.dot(x, resW_ref[...], preferred_element_type=jnp.float32)
    t = _ln(t + resB_ref[...], resG_ref[...], resBe_ref[...])
    id_ref[...] = _gelu(t)
    hw2_ref[...] = dis * jnp.dot(x, W0_ref[...],
                                 preferred_element_type=jnp.float32)


def _tc_mid_body(part_ref, dis_ref, b_ref, g_ref, be_ref, Wn_ref, hw2_ref):
    dis = dis_ref[...]
    s = jnp.concatenate([part_ref[0], part_ref[1]], axis=-1)
    s = s * dis + b_ref[...]
    h = _gelu(_ln(s, g_ref[...], be_ref[...]))
    hw2_ref[...] = dis * jnp.dot(h, Wn_ref[...],
                                 preferred_element_type=jnp.float32)


def _tc_fin_body(part_ref, dis_ref, b_ref, g_ref, be_ref, id_ref, out_ref):
    s = jnp.concatenate([part_ref[0], part_ref[1]], axis=-1)
    s = s * dis_ref[...] + b_ref[...]
    h = _gelu(_ln(s, g_ref[...], be_ref[...])) + id_ref[...]
    nrm = jnp.sqrt(jnp.sum(h * h, axis=-1, keepdims=True))
    out_ref[...] = h / jnp.maximum(nrm, 1e-8)


_G = NP // R
_spec_nd = pl.BlockSpec((R, D), lambda i: (i, 0))
_spec_n1 = pl.BlockSpec((R, 1), lambda i: (i, 0))
_spec_2n1 = pl.BlockSpec((2, R, 1), lambda i: (0, i, 0))
_spec_2nh = pl.BlockSpec((2, R, H), lambda i: (0, i, 0))
_spec_dd = pl.BlockSpec((D, D), lambda i: (0, 0))
_spec_d = pl.BlockSpec((D,), lambda i: (0,))

_f32 = jnp.float32

_tc_pre = pl.pallas_call(
    _tc_pre_body,
    grid=(_G,),
    in_specs=[_spec_nd, _spec_2n1, _spec_dd, _spec_d, _spec_d, _spec_d,
              _spec_dd],
    out_specs=[_spec_nd, _spec_nd, _spec_n1],
    out_shape=[jax.ShapeDtypeStruct((NP, D), _f32),
               jax.ShapeDtypeStruct((NP, D), _f32),
               jax.ShapeDtypeStruct((NP, 1), _f32)],
)

_tc_mid = pl.pallas_call(
    _tc_mid_body,
    grid=(_G,),
    in_specs=[_spec_2nh, _spec_n1, _spec_d, _spec_d, _spec_d, _spec_dd],
    out_specs=_spec_nd,
    out_shape=jax.ShapeDtypeStruct((NP, D), _f32),
)

_tc_fin = pl.pallas_call(
    _tc_fin_body,
    grid=(_G,),
    in_specs=[_spec_2nh, _spec_n1, _spec_d, _spec_d, _spec_d, _spec_nd],
    out_specs=_spec_nd,
    out_shape=jax.ShapeDtypeStruct((NP, D), _f32),
)


def kernel(x, edge_index, edge_weight, convW, convB, lnG, lnB, resW, resB,
           resLnG, resLnB):
    x_p = jnp.pad(x, ((0, NP - N), (0, 0)))
    row_f = jnp.pad(edge_index[0], (0, EP - E))
    col_f = jnp.pad(edge_index[1], (0, EP - E))
    w_f = jnp.pad(edge_weight, (0, EP - E))
    rows2_r = jnp.stack([row_f * 2, row_f * 2 + 1]).reshape(2, 16, CH, K)
    cols_r = col_f.reshape(16, CH, K)
    w_r = w_f.reshape(16, CH, K)
    cols_rd = col_f.reshape(32, CHD, KD)
    w_rd = w_f.reshape(32, CHD, KD)

    degp = _sc_deg(cols_rd, w_rd)[..., None]
    identity, hw2, dis = _tc_pre(x_p, degp, resW, resB, resLnG, resLnB,
                                 convW[0])
    for i in range(NC):
        hw2v = hw2.reshape(2 * NP, H)
        part = _sc_layer(hw2v, rows2_r, cols_r, w_r)
        if i < NC - 1:
            hw2 = _tc_mid(part, dis, convB[i], lnG[i], lnB[i], convW[i + 1])
        else:
            out = _tc_fin(part, dis, convB[i], lnG[i], lnB[i], identity)
    return out[:N]
